# Initial kernel scaffold; baseline (speedup 1.0000x reference)
import jax
import jax.numpy as jnp
from jax.experimental import pallas as pl


def kernel(mlvl_bboxes, mlvl_bboxes_for_nms, mlvl_scores):
    raise NotImplementedError("write your pallas kernel here")


# speedup vs baseline: 1291.3028x; 1291.3028x over previous
import functools

import jax
import jax.numpy as jnp
from jax.experimental import pallas as pl
from jax.experimental.pallas import tpu as pltpu

N = 5000
C = 3
M = 100
SCORE_THR = 0.05
NMS_THR = 0.5


def _nms_body(bb7_ref, nmsT_ref, scT_ref, out_ref, active_ref, idx_ref,
              val_ref, vld_ref):
    xc = nmsT_ref[0:1, :]
    yc = nmsT_ref[1:2, :]
    w = nmsT_ref[2:3, :]
    h = nmsT_ref[3:4, :]
    half_w = w / 2.0
    half_h = h / 2.0
    x1 = xc - half_w
    y1 = yc - half_h
    x2 = xc + half_w
    y2 = yc + half_h
    area = (x2 - x1) * (y2 - y1)

    s3 = scT_ref[0:C, :]
    active_ref[...] = (s3 > SCORE_THR).astype(jnp.float32)

    iota_n = jax.lax.broadcasted_iota(jnp.int32, (C, N), 1)
    neg_inf = jnp.float32(-jnp.inf)

    def body(i, carry):
        active = active_ref[...] > 0.5
        ms = jnp.where(active, s3, neg_inf)
        v = jnp.max(ms, axis=1, keepdims=True)
        valid = v > neg_inf
        eq = ms == v
        pos = jnp.where(eq, iota_n, N)
        idx = jnp.min(pos, axis=1, keepdims=True)
        sel = iota_n == idx

        def pick(row):
            return jnp.sum(jnp.where(sel, row, 0.0), axis=1, keepdims=True)

        sx1 = pick(x1)
        sy1 = pick(y1)
        sx2 = pick(x2)
        sy2 = pick(y2)
        s_area = (sx2 - sx1) * (sy2 - sy1)
        iw = jnp.maximum(
            jnp.minimum(sx2, x2) - jnp.maximum(sx1, x1), 0.0)
        ih = jnp.maximum(jnp.minimum(sy2, y2) - jnp.maximum(sy1, y1), 0.0)
        inter = iw * ih
        union = area + s_area - inter
        iou = inter / jnp.maximum(union, 1e-8)
        supp = (iou > NMS_THR) | sel
        active_ref[...] = jnp.where(
            active & jnp.logical_not(supp & valid), 1.0, 0.0)

        score = jnp.where(valid, v, 0.0)
        validf = valid.astype(jnp.float32)
        for c in range(C):
            idx_ref[pl.ds(i, 1), c:c + 1] = idx[c:c + 1, :]
            val_ref[pl.ds(i, 1), c:c + 1] = score[c:c + 1, :]
            vld_ref[pl.ds(i, 1), c:c + 1] = validf[c:c + 1, :]
        return carry

    jax.lax.fori_loop(0, M, body, 0)

    bb = bb7_ref[...]
    idxb = idx_ref[...]
    valb = val_ref[...]
    vldb = vld_ref[...]
    iota_mn = jax.lax.broadcasted_iota(jnp.int32, (M, N), 1)
    for c in range(C):
        idc = idxb[:, c:c + 1]
        vlc = vldb[:, c:c + 1]
        onehot = jnp.where(iota_mn == idc, vlc, 0.0)
        out7 = jnp.dot(onehot, bb, preferred_element_type=jnp.float32)
        out_ref[c * M:(c + 1) * M, 0:7] = out7
        out_ref[c * M:(c + 1) * M, 7:8] = valb[:, c:c + 1]
        out_ref[c * M:(c + 1) * M, 8:9] = vlc * float(c)


@jax.jit
def kernel(mlvl_bboxes, mlvl_bboxes_for_nms, mlvl_scores):
    nmsT = mlvl_bboxes_for_nms.T
    scT = mlvl_scores.T
    return pl.pallas_call(
        _nms_body,
        out_shape=jax.ShapeDtypeStruct((C * M, 9), jnp.float32),
        scratch_shapes=[
            pltpu.VMEM((C, N), jnp.float32),
            pltpu.VMEM((M, C), jnp.int32),
            pltpu.VMEM((M, C), jnp.float32),
            pltpu.VMEM((M, C), jnp.float32),
        ],
    )(mlvl_bboxes, nmsT, scT)
